# baseline (speedup 1.0000x reference)
import functools

import jax
import jax.numpy as jnp
from jax import lax
from jax.experimental import pallas as pl
from jax.experimental.pallas import tpu as pltpu, tpu_sc as plsc

_LANE = 128


def kernel(rois, gt_boxes):
    B, N, C = rois.shape
    _, G, _ = gt_boxes.shape
    r_t = jnp.transpose(rois, (0, 2, 1))
    g_t = jnp.transpose(gt_boxes, (0, 2, 1))

    info = plsc.get_sparse_core_info()
    NC, NS = info.num_cores, info.num_subcores
    NW = NC * NS
    nalign = (N // _LANE) * _LANE
    ntiles = nalign // _LANE
    per_t = ntiles // NW
    rem_t = ntiles - per_t * NW
    W = per_t * _LANE
    tail = N + G - nalign

    mesh = plsc.VectorSubcoreMesh(core_axis_name="c", subcore_axis_name="s")

    @functools.partial(
        pl.kernel, mesh=mesh,
        out_type=jax.ShapeDtypeStruct((B, C, N + G), rois.dtype),
        scratch_types=[
            pltpu.VMEM((B, C, W), rois.dtype),
            pltpu.VMEM((B, C, _LANE), rois.dtype),
            pltpu.VMEM((B, C, tail), rois.dtype),
        ],
    )
    def sc_concat(r_hbm, g_hbm, o_hbm, buf, rbuf, tbuf):
        wid = lax.axis_index("s") * NC + lax.axis_index("c")
        lo = wid * W
        pltpu.sync_copy(r_hbm.at[:, :, pl.ds(lo, W)], buf)
        pltpu.sync_copy(buf, o_hbm.at[:, :, pl.ds(lo, W)])

        @pl.when(wid < rem_t)
        def _():
            rlo = (per_t * NW + wid) * _LANE
            pltpu.sync_copy(r_hbm.at[:, :, pl.ds(rlo, _LANE)], rbuf)
            pltpu.sync_copy(rbuf, o_hbm.at[:, :, pl.ds(rlo, _LANE)])

        del g_hbm, tbuf

    out_t = sc_concat(r_t, g_t)
    return jnp.transpose(out_t, (0, 2, 1))


# speedup vs baseline: 10.5292x; 10.5292x over previous
import functools

import jax
import jax.numpy as jnp
from jax.experimental import pallas as pl


def _concat_body(n, r_ref, g_ref, o_ref):
    o_ref[:, :, :n] = r_ref[...]
    o_ref[:, :, n:] = g_ref[...]


def kernel(rois, gt_boxes):
    B, N, C = rois.shape
    _, G, _ = gt_boxes.shape
    r_t = jnp.transpose(rois, (0, 2, 1))
    g_t = jnp.transpose(gt_boxes, (0, 2, 1))
    body = functools.partial(_concat_body, N)
    out_t = pl.pallas_call(
        body,
        grid=(B,),
        in_specs=[
            pl.BlockSpec((1, C, N), lambda i: (i, 0, 0)),
            pl.BlockSpec((1, C, G), lambda i: (i, 0, 0)),
        ],
        out_specs=pl.BlockSpec((1, C, N + G), lambda i: (i, 0, 0)),
        out_shape=jax.ShapeDtypeStruct((B, C, N + G), rois.dtype),
    )(r_t, g_t)
    return jnp.transpose(out_t, (0, 2, 1))
